# baseline (speedup 1.0000x reference)
import jax
import jax.numpy as jnp
from jax.experimental import pallas as pl
from jax.experimental.pallas import tpu as pltpu

B, N, M, C1, C2, CO = 8, 4096, 1024, 64, 128, 128
NB = 1024
GN = N // NB
EPS_BN = 1e-5


def _pass1_kernel(uT_ref, k2x_ref, kk2_ref, u2_ref, kf_ref, uf_ref,
                  W1a_ref, W1b_ref, x1_ref, sq_ref, sacc, qacc):
    b = pl.program_id(0)
    i = pl.program_id(1)
    u = uT_ref[0]
    k2x = k2x_ref[0]
    k2 = kk2_ref[0]
    u2 = u2_ref[0]
    kf = kf_ref[0]
    uf = uf_ref[0]

    acc2 = jnp.dot(k2x, u, preferred_element_type=jnp.float32)
    d = k2 - acc2

    big = jnp.float32(3.0e38)
    vals = []
    masks = []
    for t in range(3):
        v = jnp.min(d, axis=0, keepdims=True)
        m = d == v
        d = jnp.where(m, big, d)
        masks.append(m)
        vals.append(v + u2)

    rs = [1.0 / (jnp.sqrt(jnp.maximum(v, 1e-12)) + 1e-8) for v in vals]
    rsum = rs[0] + rs[1] + rs[2]
    w = [r / rsum for r in rs]

    ST = jnp.where(masks[0], w[0],
         jnp.where(masks[1], w[1],
         jnp.where(masks[2], w[2], jnp.float32(0.0))))
    interp = jnp.dot(kf, ST, preferred_element_type=jnp.float32)

    x1 = (jnp.dot(W1a_ref[...], interp, preferred_element_type=jnp.float32)
          + jnp.dot(W1b_ref[...], uf, preferred_element_type=jnp.float32))
    x1_ref[0] = x1
    s = jnp.sum(x1, axis=1, keepdims=True)
    q = jnp.sum(x1 * x1, axis=1, keepdims=True)

    first = (b == 0) & (i == 0)
    last = (b == B - 1) & (i == GN - 1)

    @pl.when(first)
    def _():
        sacc[...] = s
        qacc[...] = q

    @pl.when(~first)
    def _():
        sacc[...] += s
        qacc[...] += q

    @pl.when(last)
    def _():
        sq_ref[:, 0:1] = sacc[...]
        sq_ref[:, 1:2] = qacc[...]


def _pass2_kernel(x1_ref, a1_ref, c1_ref, W2_ref, x2_ref, sq_ref, sacc, qacc):
    b = pl.program_id(0)
    i = pl.program_id(1)
    h = jnp.maximum(x1_ref[0] * a1_ref[...] + c1_ref[...], 0.0)
    x2 = jnp.dot(W2_ref[...], h, preferred_element_type=jnp.float32)
    x2_ref[0] = x2
    s = jnp.sum(x2, axis=1, keepdims=True)
    q = jnp.sum(x2 * x2, axis=1, keepdims=True)

    first = (b == 0) & (i == 0)
    last = (b == B - 1) & (i == GN - 1)

    @pl.when(first)
    def _():
        sacc[...] = s
        qacc[...] = q

    @pl.when(~first)
    def _():
        sacc[...] += s
        qacc[...] += q

    @pl.when(last)
    def _():
        sq_ref[:, 0:1] = sacc[...]
        sq_ref[:, 1:2] = qacc[...]


def _pass3_kernel(x2_ref, a2_ref, c2_ref, o_ref):
    o_ref[0] = jnp.maximum(x2_ref[0] * a2_ref[...] + c2_ref[...], 0.0)


def _bn_affine(sq, g, b):
    cnt = float(B * N)
    mean = sq[:, 0] / cnt
    var = sq[:, 1] / cnt - mean * mean
    a = g / jnp.sqrt(var + EPS_BN)
    c = b - a * mean
    return a[:, None], c[:, None]


def kernel(unknown, known, unknow_feats, known_feats, W1, g1, b1, W2, g2, b2):
    f32 = jnp.float32
    uT = unknown.transpose(0, 2, 1)
    u2 = jnp.sum(unknown * unknown, axis=2)[:, None, :]
    k2x = 2.0 * known
    kk2 = jnp.sum(known * known, axis=2, keepdims=True)
    W1a = W1[:, :C2]
    W1b = W1[:, C2:]

    x1, sq1 = pl.pallas_call(
        _pass1_kernel,
        grid=(B, GN),
        in_specs=[
            pl.BlockSpec((1, 3, NB), lambda b, i: (b, 0, i)),
            pl.BlockSpec((1, M, 3), lambda b, i: (b, 0, 0)),
            pl.BlockSpec((1, M, 1), lambda b, i: (b, 0, 0)),
            pl.BlockSpec((1, 1, NB), lambda b, i: (b, 0, i)),
            pl.BlockSpec((1, C2, M), lambda b, i: (b, 0, 0)),
            pl.BlockSpec((1, C1, NB), lambda b, i: (b, 0, i)),
            pl.BlockSpec((CO, C2), lambda b, i: (0, 0)),
            pl.BlockSpec((CO, C1), lambda b, i: (0, 0)),
        ],
        out_specs=[
            pl.BlockSpec((1, CO, NB), lambda b, i: (b, 0, i)),
            pl.BlockSpec((CO, 2), lambda b, i: (0, 0)),
        ],
        out_shape=[
            jax.ShapeDtypeStruct((B, CO, N), f32),
            jax.ShapeDtypeStruct((CO, 2), f32),
        ],
        scratch_shapes=[pltpu.VMEM((CO, 1), f32), pltpu.VMEM((CO, 1), f32)],
    )(uT, k2x, kk2, u2, known_feats, unknow_feats, W1a, W1b)

    a1, c1 = _bn_affine(sq1, g1, b1)

    x2, sq2 = pl.pallas_call(
        _pass2_kernel,
        grid=(B, GN),
        in_specs=[
            pl.BlockSpec((1, CO, NB), lambda b, i: (b, 0, i)),
            pl.BlockSpec((CO, 1), lambda b, i: (0, 0)),
            pl.BlockSpec((CO, 1), lambda b, i: (0, 0)),
            pl.BlockSpec((CO, CO), lambda b, i: (0, 0)),
        ],
        out_specs=[
            pl.BlockSpec((1, CO, NB), lambda b, i: (b, 0, i)),
            pl.BlockSpec((CO, 2), lambda b, i: (0, 0)),
        ],
        out_shape=[
            jax.ShapeDtypeStruct((B, CO, N), f32),
            jax.ShapeDtypeStruct((CO, 2), f32),
        ],
        scratch_shapes=[pltpu.VMEM((CO, 1), f32), pltpu.VMEM((CO, 1), f32)],
    )(x1, a1, c1, W2)

    a2, c2 = _bn_affine(sq2, g2, b2)

    out = pl.pallas_call(
        _pass3_kernel,
        grid=(B,),
        in_specs=[
            pl.BlockSpec((1, CO, N), lambda b: (b, 0, 0)),
            pl.BlockSpec((CO, 1), lambda b: (0, 0)),
            pl.BlockSpec((CO, 1), lambda b: (0, 0)),
        ],
        out_specs=pl.BlockSpec((1, CO, N), lambda b: (b, 0, 0)),
        out_shape=jax.ShapeDtypeStruct((B, CO, N), f32),
    )(x2, a2, c2)
    return out
